# baseline (speedup 1.0000x reference)
import math

import jax
import jax.numpy as jnp
from jax.experimental import pallas as pl

_NLAYERS = 4
_LAMDA = 0.5
_ALPHA = 0.1
_EDGE = 0.99999
_DIAS_PER_STEP = 10

_ACOS_C = (1.5707288, -0.2121144, 0.0742610, -0.0187293)


def _acos(x):
    ax = jnp.abs(x)
    p = _ACOS_C[3]
    for c in (_ACOS_C[2], _ACOS_C[1], _ACOS_C[0]):
        p = p * ax + c
    r = jnp.sqrt(1.0 - ax) * p
    return jnp.where(x < 0, math.pi - r, r)


def _bdot(x, y):
    return jax.lax.dot_general(x, y, (((2,), (1,)), ((0,), (0,))),
                               preferred_element_type=jnp.float32)


def _wdot(x, w):
    return jax.lax.dot_general(x, w, (((2,), (0,)), ((), ())),
                               preferred_element_type=jnp.float32)


def _gcn_kernel(a_ref, v_ref, l_ref, wa_ref, ba_ref, wv_ref, bv_ref, wl_ref,
                bl_ref, conv_ref, out_ref):
    a = a_ref[...]
    v = v_ref[...]
    l = l_ref[...]

    nh = conv_ref.shape[2]
    r0 = jax.lax.broadcasted_iota(jnp.int32, (nh, nh), 0)
    r1 = jax.lax.broadcasted_iota(jnp.int32, (nh, nh), 1)
    eye = jnp.where(r0 == r1, 1.0, 0.0).astype(jnp.float32)
    w12s = []
    for i in range(_NLAYERS):
        theta = math.log(_LAMDA / (i + 1) + 1.0)
        w1 = theta * conv_ref[i, :nh, :] + (1.0 - theta) * (1.0 - _ALPHA) * eye
        w2 = theta * conv_ref[i, nh:, :] + (1.0 - theta) * _ALPHA * eye
        w12s.append(jnp.concatenate([w1, w2], axis=0))

    def sim_block(x):
        vec_len = jnp.sqrt(jnp.sum(x * x, axis=2, keepdims=True))
        nt = x / vec_len
        cos = jax.lax.dot_general(nt, nt, (((2,), (2,)), ((0,), (0,))),
                                  preferred_element_type=jnp.float32) * _EDGE
        return _acos(cos)

    sims = [sim_block(a), sim_block(v), sim_block(l)]

    dis = []
    dis2 = []
    for s in sims:
        rs = jnp.sum(s, axis=2, keepdims=True) + 2.0 * _EDGE
        dis.append(rs ** -0.5)
        dis2.append(1.0 / rs)

    def fc(x, w_ref, b_ref):
        y = jax.lax.dot_general(x, w_ref[...], (((2,), (1,)), ((), ())),
                                preferred_element_type=jnp.float32)
        return jax.nn.relu(y + b_ref[...])

    h0 = [fc(a, wa_ref, ba_ref), fc(v, wv_ref, bv_ref), fc(l, wl_ref, bl_ref)]
    z0 = [dis[m] * h0[m] for m in range(3)]
    z = list(z0)

    for i in range(_NLAYERS):
        last = i == _NLAYERS - 1
        w12 = w12s[i]
        zsum = z[0] + z[1] + z[2]
        new = []
        for m in range(3):
            u = _bdot(sims[m], z[m]) + _EDGE * (zsum - z[m])
            support = jnp.concatenate(
                [(dis[m] if last else dis2[m]) * u,
                 h0[m] if last else z0[m]], axis=2)
            new.append(jax.nn.relu(_wdot(support, w12)))
        z = new

    out_ref[...] = jnp.concatenate([l, z[0], z[1], z[2]], axis=2)


def kernel(a, v, l, dia_len, topicLabel, fc_a_w, fc_a_b, fc_v_w, fc_v_b,
           fc_l_w, fc_l_b, conv_w):
    n, nfeat = l.shape
    ndia = dia_len.shape[0]
    dlen = n // ndia
    nhid = fc_a_w.shape[0]
    grid = ndia // _DIAS_PER_STEP

    a3 = a.reshape(ndia, dlen, nfeat)
    v3 = v.reshape(ndia, dlen, nfeat)
    l3 = l.reshape(ndia, dlen, nfeat)
    ba = fc_a_b.reshape(1, nhid)
    bv = fc_v_b.reshape(1, nhid)
    bl = fc_l_b.reshape(1, nhid)

    feat_spec = pl.BlockSpec((_DIAS_PER_STEP, dlen, nfeat),
                             lambda d: (d, 0, 0))
    w_spec = pl.BlockSpec((nhid, nfeat), lambda d: (0, 0))
    b_spec = pl.BlockSpec((1, nhid), lambda d: (0, 0))
    conv_spec = pl.BlockSpec((_NLAYERS, 2 * nhid, nhid), lambda d: (0, 0, 0))

    out = pl.pallas_call(
        _gcn_kernel,
        grid=(grid,),
        in_specs=[feat_spec, feat_spec, feat_spec, w_spec, b_spec, w_spec,
                  b_spec, w_spec, b_spec, conv_spec],
        out_specs=pl.BlockSpec((_DIAS_PER_STEP, dlen, nfeat + 3 * nhid),
                               lambda d: (d, 0, 0)),
        out_shape=jax.ShapeDtypeStruct((ndia, dlen, nfeat + 3 * nhid),
                                       jnp.float32),
    )(a3, v3, l3, fc_a_w, ba, fc_v_w, bv, fc_l_w, bl, conv_w)

    return out.reshape(n, nfeat + 3 * nhid)


# speedup vs baseline: 1.0425x; 1.0425x over previous
import math

import jax
import jax.numpy as jnp
from jax.experimental import pallas as pl

_NLAYERS = 4
_LAMDA = 0.5
_ALPHA = 0.1
_EDGE = 0.99999
_DIAS_PER_STEP = 5

_ACOS_C = (1.5707288, -0.2121144, 0.0742610, -0.0187293)


def _acos(x):
    ax = jnp.abs(x)
    p = _ACOS_C[3]
    for c in (_ACOS_C[2], _ACOS_C[1], _ACOS_C[0]):
        p = p * ax + c
    r = jnp.sqrt(1.0 - ax) * p
    return jnp.where(x < 0, math.pi - r, r)


def _bdot(x, y):
    return jax.lax.dot_general(x, y, (((2,), (1,)), ((0,), (0,))),
                               preferred_element_type=jnp.float32)


def _wdot(x, w):
    return jax.lax.dot_general(x, w, (((2,), (0,)), ((), ())),
                               preferred_element_type=jnp.float32)


def _gcn_kernel(a_ref, v_ref, l_ref, wa_ref, ba_ref, wv_ref, bv_ref, wl_ref,
                bl_ref, conv_ref, out_ref):
    a = a_ref[...]
    v = v_ref[...]
    l = l_ref[...]

    nh = conv_ref.shape[2]
    r0 = jax.lax.broadcasted_iota(jnp.int32, (nh, nh), 0)
    r1 = jax.lax.broadcasted_iota(jnp.int32, (nh, nh), 1)
    eye = jnp.where(r0 == r1, 1.0, 0.0).astype(jnp.float32)
    w12s = []
    for i in range(_NLAYERS):
        theta = math.log(_LAMDA / (i + 1) + 1.0)
        w1 = theta * conv_ref[i, :nh, :] + (1.0 - theta) * (1.0 - _ALPHA) * eye
        w2 = theta * conv_ref[i, nh:, :] + (1.0 - theta) * _ALPHA * eye
        w12s.append(jnp.concatenate([w1, w2], axis=0))

    def sim_block(x):
        inv = jax.lax.rsqrt(jnp.sum(x * x, axis=2, keepdims=True))
        nt = x * (inv * math.sqrt(_EDGE))
        cos = jax.lax.dot_general(nt, nt, (((2,), (2,)), ((0,), (0,))),
                                  preferred_element_type=jnp.float32)
        return _acos(cos)

    sims = [sim_block(a), sim_block(v), sim_block(l)]

    dis = []
    dis2 = []
    for s in sims:
        rs = jnp.sum(s, axis=2, keepdims=True) + 2.0 * _EDGE
        dis.append(rs ** -0.5)
        dis2.append(1.0 / rs)

    def fc(x, w_ref, b_ref):
        y = jax.lax.dot_general(x, w_ref[...], (((2,), (1,)), ((), ())),
                                preferred_element_type=jnp.float32)
        return jax.nn.relu(y + b_ref[...])

    h0 = [fc(a, wa_ref, ba_ref), fc(v, wv_ref, bv_ref), fc(l, wl_ref, bl_ref)]
    z0 = [dis[m] * h0[m] for m in range(3)]
    z = list(z0)

    for i in range(_NLAYERS):
        last = i == _NLAYERS - 1
        w12 = w12s[i]
        zsum = z[0] + z[1] + z[2]
        new = []
        for m in range(3):
            u = _bdot(sims[m], z[m]) + _EDGE * (zsum - z[m])
            support = jnp.concatenate(
                [(dis[m] if last else dis2[m]) * u,
                 h0[m] if last else z0[m]], axis=2)
            new.append(jax.nn.relu(_wdot(support, w12)))
        z = new

    out_ref[...] = jnp.concatenate([l, z[0], z[1], z[2]], axis=2)


def kernel(a, v, l, dia_len, topicLabel, fc_a_w, fc_a_b, fc_v_w, fc_v_b,
           fc_l_w, fc_l_b, conv_w):
    n, nfeat = l.shape
    ndia = dia_len.shape[0]
    dlen = n // ndia
    nhid = fc_a_w.shape[0]
    grid = ndia // _DIAS_PER_STEP

    a3 = a.reshape(ndia, dlen, nfeat)
    v3 = v.reshape(ndia, dlen, nfeat)
    l3 = l.reshape(ndia, dlen, nfeat)
    ba = fc_a_b.reshape(1, nhid)
    bv = fc_v_b.reshape(1, nhid)
    bl = fc_l_b.reshape(1, nhid)

    feat_spec = pl.BlockSpec((_DIAS_PER_STEP, dlen, nfeat),
                             lambda d: (d, 0, 0))
    w_spec = pl.BlockSpec((nhid, nfeat), lambda d: (0, 0))
    b_spec = pl.BlockSpec((1, nhid), lambda d: (0, 0))
    conv_spec = pl.BlockSpec((_NLAYERS, 2 * nhid, nhid), lambda d: (0, 0, 0))

    out = pl.pallas_call(
        _gcn_kernel,
        grid=(grid,),
        in_specs=[feat_spec, feat_spec, feat_spec, w_spec, b_spec, w_spec,
                  b_spec, w_spec, b_spec, conv_spec],
        out_specs=pl.BlockSpec((_DIAS_PER_STEP, dlen, nfeat + 3 * nhid),
                               lambda d: (d, 0, 0)),
        out_shape=jax.ShapeDtypeStruct((ndia, dlen, nfeat + 3 * nhid),
                                       jnp.float32),
    )(a3, v3, l3, fc_a_w, ba, fc_v_w, bv, fc_l_w, bl, conv_w)

    return out.reshape(n, nfeat + 3 * nhid)


# speedup vs baseline: 1.0538x; 1.0109x over previous
import math

import jax
import jax.numpy as jnp
from jax.experimental import pallas as pl

_NLAYERS = 4
_LAMDA = 0.5
_ALPHA = 0.1
_EDGE = 0.99999
_DIAS_PER_STEP = 5

_ACOS_C = (1.5707288, -0.2121144, 0.0742610, -0.0187293)


def _acos(x):
    ax = jnp.abs(x)
    p = _ACOS_C[3]
    for c in (_ACOS_C[2], _ACOS_C[1], _ACOS_C[0]):
        p = p * ax + c
    r = jnp.sqrt(1.0 - ax) * p
    return jnp.where(x < 0, math.pi - r, r)


def _bdot(x, y):
    return jax.lax.dot_general(x, y, (((2,), (1,)), ((0,), (0,))),
                               preferred_element_type=jnp.float32)


def _wdot(x, w):
    return jax.lax.dot_general(x, w, (((2,), (0,)), ((), ())),
                               preferred_element_type=jnp.float32)


def _gcn_kernel(a_ref, v_ref, l_ref, wa_ref, ba_ref, wv_ref, bv_ref, wl_ref,
                bl_ref, conv_ref, out_ref):
    a = a_ref[...]
    v = v_ref[...]
    l = l_ref[...]

    nh = conv_ref.shape[2]
    r0 = jax.lax.broadcasted_iota(jnp.int32, (nh, nh), 0)
    r1 = jax.lax.broadcasted_iota(jnp.int32, (nh, nh), 1)
    eye = jnp.where(r0 == r1, 1.0, 0.0).astype(jnp.float32)
    w12s = []
    for i in range(_NLAYERS):
        theta = math.log(_LAMDA / (i + 1) + 1.0)
        w1 = theta * conv_ref[i, :nh, :] + (1.0 - theta) * (1.0 - _ALPHA) * eye
        w2 = theta * conv_ref[i, nh:, :] + (1.0 - theta) * _ALPHA * eye
        w12s.append(jnp.concatenate([w1, w2], axis=0))

    def sim_block(x):
        inv = jax.lax.rsqrt(jnp.sum(x * x, axis=2, keepdims=True))
        nt = x * (inv * math.sqrt(_EDGE))
        cos = jax.lax.dot_general(nt, nt, (((2,), (2,)), ((0,), (0,))),
                                  preferred_element_type=jnp.float32)
        return _acos(cos)

    sims = [sim_block(a), sim_block(v), sim_block(l)]

    dis = []
    dis2 = []
    for s in sims:
        rs = jnp.sum(s, axis=2, keepdims=True) + 2.0 * _EDGE
        dis.append(rs ** -0.5)
        dis2.append(1.0 / rs)

    def fc(x, w_ref, b_ref):
        y = jax.lax.dot_general(x, w_ref[...], (((2,), (1,)), ((), ())),
                                preferred_element_type=jnp.float32)
        return jax.nn.relu(y + b_ref[...])

    h0 = [fc(a, wa_ref, ba_ref), fc(v, wv_ref, bv_ref), fc(l, wl_ref, bl_ref)]
    z0 = [dis[m] * h0[m] for m in range(3)]
    z = list(z0)

    for i in range(_NLAYERS):
        last = i == _NLAYERS - 1
        w12 = w12s[i]
        zpair = [z[1] + z[2], z[0] + z[2], z[0] + z[1]]
        new = []
        for m in range(3):
            u = _bdot(sims[m], z[m]) + _EDGE * zpair[m]
            support = jnp.concatenate(
                [(dis[m] if last else dis2[m]) * u,
                 h0[m] if last else z0[m]], axis=2)
            new.append(jax.nn.relu(_wdot(support, w12)))
        z = new

    out_ref[...] = jnp.concatenate([l, z[0], z[1], z[2]], axis=2)


def kernel(a, v, l, dia_len, topicLabel, fc_a_w, fc_a_b, fc_v_w, fc_v_b,
           fc_l_w, fc_l_b, conv_w):
    n, nfeat = l.shape
    ndia = dia_len.shape[0]
    dlen = n // ndia
    nhid = fc_a_w.shape[0]
    grid = ndia // _DIAS_PER_STEP

    a3 = a.reshape(ndia, dlen, nfeat)
    v3 = v.reshape(ndia, dlen, nfeat)
    l3 = l.reshape(ndia, dlen, nfeat)
    ba = fc_a_b.reshape(1, nhid)
    bv = fc_v_b.reshape(1, nhid)
    bl = fc_l_b.reshape(1, nhid)

    feat_spec = pl.BlockSpec((_DIAS_PER_STEP, dlen, nfeat),
                             lambda d: (d, 0, 0))
    w_spec = pl.BlockSpec((nhid, nfeat), lambda d: (0, 0))
    b_spec = pl.BlockSpec((1, nhid), lambda d: (0, 0))
    conv_spec = pl.BlockSpec((_NLAYERS, 2 * nhid, nhid), lambda d: (0, 0, 0))

    out = pl.pallas_call(
        _gcn_kernel,
        grid=(grid,),
        in_specs=[feat_spec, feat_spec, feat_spec, w_spec, b_spec, w_spec,
                  b_spec, w_spec, b_spec, conv_spec],
        out_specs=pl.BlockSpec((_DIAS_PER_STEP, dlen, nfeat + 3 * nhid),
                               lambda d: (d, 0, 0)),
        out_shape=jax.ShapeDtypeStruct((ndia, dlen, nfeat + 3 * nhid),
                                       jnp.float32),
    )(a3, v3, l3, fc_a_w, ba, fc_v_w, bv, fc_l_w, bl, conv_w)

    return out.reshape(n, nfeat + 3 * nhid)
